# baseline (speedup 1.0000x reference)
import functools

import jax
import jax.numpy as jnp
from jax import lax
from jax.experimental import pallas as pl
from jax.experimental.pallas import tpu as pltpu
from jax.experimental.pallas import tpu_sc as plsc

_B = 4096
_L = 200
_D = 32
_NC = 2
_NS = 16
_NW = _NC * _NS
_BPW = _B // _NW
_FV = 15000

_GC = 8
_HB = 4
_DPT = _D // _GC
_BPH = _B // _HB
_CHB = 64
_NCH = _BPH // _CHB

_mesh = plsc.VectorSubcoreMesh(core_axis_name="c", subcore_axis_name="s")


def _sc_body(featsT_hbm, ftabT_hbm, fsumT_hbm, tab_v, fT_v, out_v, sems):
    wid = lax.axis_index("s") * _NC + lax.axis_index("c")
    g = wid // _HB
    h = wid % _HB
    pltpu.sync_copy(ftabT_hbm.at[pl.ds(_DPT * g, _DPT), :], tab_v)

    def issue(k, slot):
        kk = jnp.minimum(k, _NCH - 1)
        pltpu.async_copy(
            featsT_hbm.at[:, pl.ds(_BPH * h + _CHB * kk, _CHB)],
            fT_v.at[slot], sems.at[slot])

    def drain(slot):
        pltpu.make_async_copy(featsT_hbm.at[:, pl.ds(0, _CHB)],
                              fT_v.at[slot], sems.at[slot]).wait()

    def process(k, slot):
        zeros = jnp.zeros((16,), jnp.float32)

        @plsc.parallel_loop(0, _L, step=1, unroll=4, carry=(zeros,) * 16)
        def red(l, accs):
            a = list(accs)
            for lg in range(4):
                idx16 = fT_v[slot, l, pl.ds(16 * lg, 16)]
                for d in range(_DPT):
                    v = plsc.load_gather(tab_v.at[d], [idx16])
                    a[_DPT * lg + d] = a[_DPT * lg + d] + v
            return tuple(a)

        acc = red
        for lg in range(4):
            for d in range(_DPT):
                out_v[d, pl.ds(_CHB * k + 16 * lg, 16)] = acc[_DPT * lg + d]

    issue(0, 0)

    def pair(p, carry):
        k0 = 2 * p
        issue(k0 + 1, 1)
        drain(0)
        process(k0, 0)
        issue(k0 + 2, 0)
        drain(1)
        process(k0 + 1, 1)
        return carry

    lax.fori_loop(0, _NCH // 2, pair, 0)
    drain(0)
    pltpu.sync_copy(
        out_v, fsumT_hbm.at[pl.ds(_DPT * g, _DPT), pl.ds(_BPH * h, _BPH)])


_sc_pool = functools.partial(
    pl.kernel,
    out_type=jax.ShapeDtypeStruct((_D, _B), jnp.float32),
    mesh=_mesh,
    scratch_types=[
        pltpu.VMEM((_DPT, _FV), jnp.float32),
        pltpu.VMEM((2, _L, _CHB), jnp.int32),
        pltpu.VMEM((_DPT, _BPH), jnp.float32),
        pltpu.SemaphoreType.DMA((2,)),
    ],
    compiler_params=pltpu.CompilerParams(use_tc_tiling_on_sc=False,
                                         needs_layout_passes=False),
)(_sc_body)


def _sc_node_body(nidx_hbm, ntab_hbm, nrow_hbm, nidx_v, nrow_v, sem_n):
    wid = lax.axis_index("s") * _NC + lax.axis_index("c")
    base = wid * _BPW
    pltpu.sync_copy(nidx_hbm.at[pl.ds(base, _BPW)], nidx_v)
    nst = 8
    npc = _BPW // nst
    cps = [pltpu.async_copy(ntab_hbm.at[nidx_v.at[pl.ds(i * npc, npc)]],
                            nrow_v.at[pl.ds(i * npc, npc), :], sem_n)
           for i in range(nst)]
    for cp in cps:
        cp.wait()
    pltpu.sync_copy(nrow_v, nrow_hbm.at[pl.ds(base, _BPW), :])


_sc_node = functools.partial(
    pl.kernel,
    out_type=jax.ShapeDtypeStruct((_B, _D), jnp.float32),
    mesh=_mesh,
    scratch_types=[
        pltpu.VMEM((_BPW,), jnp.int32),
        pltpu.VMEM((_BPW, _D), jnp.float32),
        pltpu.SemaphoreType.DMA,
    ],
    compiler_params=pltpu.CompilerParams(use_tc_tiling_on_sc=False),
)(_sc_node_body)


def _tc_body(fsumt_ref, nrow_ref, fw_ref, fb_ref, nw_ref, nb_ref, out_ref):
    fo = lax.dot_general(fsumt_ref[...], fw_ref[...], (((0,), (1,)), ((), ())),
                         preferred_element_type=jnp.float32)
    no = lax.dot_general(nrow_ref[...], nw_ref[...], (((1,), (1,)), ((), ())),
                         preferred_element_type=jnp.float32)
    out_ref[:, 0:_D] = fo * (1.0 / _L) + fb_ref[...]
    out_ref[:, _D:2 * _D] = no + nb_ref[...]


def kernel(ids, feats, layer_idx, node_table, node_fc_w, node_fc_b,
           feat_table, feat_fc_w, feat_fc_b):
    n_nodes = node_table.shape[0] - 1
    idx = jnp.where(layer_idx > 0, ids,
                    jnp.full_like(ids, n_nodes)).astype(jnp.int32)
    featsT = jnp.swapaxes(feats.astype(jnp.int32), 0, 1)
    ftabT = jnp.swapaxes(feat_table, 0, 1)
    fsumT = jnp.zeros((_D, _B), jnp.float32)
    nrow = _sc_node(idx, node_table)
    out = pl.pallas_call(
        _tc_body,
        out_shape=jax.ShapeDtypeStruct((_B, 2 * _D), jnp.float32),
    )(fsumT, nrow, feat_fc_w, feat_fc_b.reshape(1, _D),
      node_fc_w, node_fc_b.reshape(1, _D))
    return out


# speedup vs baseline: 1.1458x; 1.1458x over previous
import functools

import jax
import jax.numpy as jnp
from jax import lax
from jax.experimental import pallas as pl
from jax.experimental.pallas import tpu as pltpu
from jax.experimental.pallas import tpu_sc as plsc

_B = 4096
_L = 200
_D = 32
_NC = 2
_NS = 16
_NW = _NC * _NS
_BPW = _B // _NW
_FV = 15000

_GC = 8
_HB = 4
_DPT = _D // _GC
_BPH = _B // _HB
_CHB = 64
_NCH = _BPH // _CHB

_mesh = plsc.VectorSubcoreMesh(core_axis_name="c", subcore_axis_name="s")


def _sc_body(featsT_hbm, ftabT_hbm, fsumT_hbm, tab_v, fT_v, out_v, sems):
    wid = lax.axis_index("s") * _NC + lax.axis_index("c")
    g = wid // _HB
    h = wid % _HB
    pltpu.sync_copy(ftabT_hbm.at[pl.ds(_DPT * g, _DPT), :], tab_v)

    def issue(k, slot):
        kk = jnp.minimum(k, _NCH - 1)
        pltpu.async_copy(
            featsT_hbm.at[:, pl.ds(_BPH * h + _CHB * kk, _CHB)],
            fT_v.at[slot], sems.at[slot])

    def drain(slot):
        pltpu.make_async_copy(featsT_hbm.at[:, pl.ds(0, _CHB)],
                              fT_v.at[slot], sems.at[slot]).wait()

    def process(k, slot):
        zeros = jnp.zeros((16,), jnp.float32)

        @plsc.parallel_loop(0, _L, step=1, unroll=4, carry=(zeros,) * 16)
        def red(l, accs):
            a = list(accs)
            for lg in range(4):
                idx16 = fT_v[slot, l, pl.ds(16 * lg, 16)]
                for d in range(_DPT):
                    v = plsc.load_gather(tab_v.at[d], [idx16])
                    a[_DPT * lg + d] = a[_DPT * lg + d] + v
            return tuple(a)

        acc = red
        for lg in range(4):
            for d in range(_DPT):
                out_v[d, pl.ds(_CHB * k + 16 * lg, 16)] = acc[_DPT * lg + d]

    issue(0, 0)

    def pair(p, carry):
        k0 = 2 * p
        issue(k0 + 1, 1)
        drain(0)
        process(k0, 0)
        issue(k0 + 2, 0)
        drain(1)
        process(k0 + 1, 1)
        return carry

    lax.fori_loop(0, _NCH // 2, pair, 0)
    drain(0)
    pltpu.sync_copy(
        out_v, fsumT_hbm.at[pl.ds(_DPT * g, _DPT), pl.ds(_BPH * h, _BPH)])


_sc_pool = functools.partial(
    pl.kernel,
    out_type=jax.ShapeDtypeStruct((_D, _B), jnp.float32),
    mesh=_mesh,
    scratch_types=[
        pltpu.VMEM((_DPT, _FV), jnp.float32),
        pltpu.VMEM((2, _L, _CHB), jnp.int32),
        pltpu.VMEM((_DPT, _BPH), jnp.float32),
        pltpu.SemaphoreType.DMA((2,)),
    ],
    compiler_params=pltpu.CompilerParams(use_tc_tiling_on_sc=False,
                                         needs_layout_passes=False),
)(_sc_body)


_NV = 100001
_NVP = 100096


def _nz_body(x_ref, out_ref):
    for r in range(8):
        out_ref[pl.ds(r * _NVP, _NV)] = x_ref[r, :]


def _node_zform(node_t):
    return pl.pallas_call(
        _nz_body,
        grid=(_D // 8,),
        in_specs=[pl.BlockSpec((8, _NV), lambda d: (d, 0))],
        out_specs=pl.BlockSpec((8 * _NVP,), lambda d: (d,)),
        out_shape=jax.ShapeDtypeStruct((_D * _NVP,), jnp.float32),
    )(node_t)


def _sc_node_body(nidx_hbm, zq_hbm, nrowT_hbm, nidx_v, zrow_v, out_v):
    wid = lax.axis_index("s") * _NC + lax.axis_index("c")
    pltpu.sync_copy(zq_hbm.at[wid, :], zrow_v)
    pltpu.sync_copy(nidx_hbm, nidx_v)

    @plsc.parallel_loop(0, _B // 16, step=1, unroll=4)
    def gat(i):
        idx16 = nidx_v[pl.ds(16 * i, 16)]
        out_v[pl.ds(16 * i, 16)] = plsc.load_gather(zrow_v, [idx16])

    pltpu.sync_copy(out_v, nrowT_hbm.at[wid, :])


_sc_node = functools.partial(
    pl.kernel,
    out_type=jax.ShapeDtypeStruct((_D, _B), jnp.float32),
    mesh=_mesh,
    scratch_types=[
        pltpu.VMEM((_B,), jnp.int32),
        pltpu.VMEM((_NVP,), jnp.float32),
        pltpu.VMEM((_B,), jnp.float32),
    ],
    compiler_params=pltpu.CompilerParams(use_tc_tiling_on_sc=False,
                                         needs_layout_passes=False),
)(_sc_node_body)


def _tc_body(fsumt_ref, nrow_ref, fw_ref, fb_ref, nw_ref, nb_ref, out_ref):
    fo = lax.dot_general(fsumt_ref[...], fw_ref[...], (((0,), (1,)), ((), ())),
                         preferred_element_type=jnp.float32)
    no = lax.dot_general(nrow_ref[...], nw_ref[...], (((0,), (1,)), ((), ())),
                         preferred_element_type=jnp.float32)
    out_ref[:, 0:_D] = fo * (1.0 / _L) + fb_ref[...]
    out_ref[:, _D:2 * _D] = no + nb_ref[...]


def kernel(ids, feats, layer_idx, node_table, node_fc_w, node_fc_b,
           feat_table, feat_fc_w, feat_fc_b):
    n_nodes = node_table.shape[0] - 1
    idx = jnp.where(layer_idx > 0, ids,
                    jnp.full_like(ids, n_nodes)).astype(jnp.int32)
    featsT = jnp.swapaxes(feats.astype(jnp.int32), 0, 1)
    ftabT = jnp.swapaxes(feat_table, 0, 1)
    node_t = jnp.swapaxes(node_table, 0, 1)
    zq = _node_zform(node_t).reshape(_D, _NVP)
    fsumT = _sc_pool(featsT, ftabT)
    nrow = _sc_node(idx, zq)
    out = pl.pallas_call(
        _tc_body,
        out_shape=jax.ShapeDtypeStruct((_B, 2 * _D), jnp.float32),
    )(fsumT, nrow, feat_fc_w, feat_fc_b.reshape(1, _D),
      node_fc_w, node_fc_b.reshape(1, _D))
    return out


# speedup vs baseline: 1.4107x; 1.2312x over previous
import functools

import jax
import jax.numpy as jnp
from jax import lax
from jax.experimental import pallas as pl
from jax.experimental.pallas import tpu as pltpu
from jax.experimental.pallas import tpu_sc as plsc

_B = 4096
_L = 200
_D = 32
_NC = 2
_NS = 16
_NW = _NC * _NS
_BPW = _B // _NW
_FV = 15000

_GC = 8
_HB = 4
_DPT = _D // _GC
_BPH = _B // _HB
_CHB = 64
_NCH = _BPH // _CHB

_mesh = plsc.VectorSubcoreMesh(core_axis_name="c", subcore_axis_name="s")


def _sc_body(featsT_hbm, ftabT_hbm, fsumT_hbm, tab_v, fT_v, out_v, sems):
    wid = lax.axis_index("s") * _NC + lax.axis_index("c")
    g = wid // _HB
    h = wid % _HB
    pltpu.sync_copy(ftabT_hbm.at[pl.ds(2 * g, 2), :], tab_v)

    def issue(k, slot):
        kk = jnp.minimum(k, _NCH - 1)
        pltpu.async_copy(
            featsT_hbm.at[:, pl.ds(_BPH * h + _CHB * kk, _CHB)],
            fT_v.at[slot], sems.at[slot])

    def drain(slot):
        pltpu.make_async_copy(featsT_hbm.at[:, pl.ds(0, _CHB)],
                              fT_v.at[slot], sems.at[slot]).wait()

    def process(k, slot):
        zeros = jnp.zeros((16,), jnp.float32)

        @plsc.parallel_loop(0, _L, step=1, unroll=4, carry=(zeros,) * 16)
        def red(l, accs):
            a = list(accs)
            for lg in range(4):
                idx16 = fT_v[slot, l, pl.ds(16 * lg, 16)]
                for p in range(2):
                    w = plsc.load_gather(tab_v.at[p], [idx16])
                    lo, hi = plsc.unpack(plsc.bitcast(w, jnp.bfloat16),
                                         format=plsc.PackFormat.INTERLEAVED)
                    a[_DPT * lg + 2 * p] = a[_DPT * lg + 2 * p] + lo
                    a[_DPT * lg + 2 * p + 1] = a[_DPT * lg + 2 * p + 1] + hi
            return tuple(a)

        acc = red
        for lg in range(4):
            for d in range(_DPT):
                out_v[d, pl.ds(_CHB * k + 16 * lg, 16)] = acc[_DPT * lg + d]

    issue(0, 0)

    def pair(p, carry):
        k0 = 2 * p
        issue(k0 + 1, 1)
        drain(0)
        process(k0, 0)
        issue(k0 + 2, 0)
        drain(1)
        process(k0 + 1, 1)
        return carry

    lax.fori_loop(0, _NCH // 2, pair, 0)
    drain(0)
    pltpu.sync_copy(
        out_v, fsumT_hbm.at[pl.ds(_DPT * g, _DPT), pl.ds(_BPH * h, _BPH)])


_sc_pool = functools.partial(
    pl.kernel,
    out_type=jax.ShapeDtypeStruct((_D, _B), jnp.float32),
    mesh=_mesh,
    scratch_types=[
        pltpu.VMEM((2, _FV), jnp.int32),
        pltpu.VMEM((2, _L, _CHB), jnp.int32),
        pltpu.VMEM((_DPT, _BPH), jnp.float32),
        pltpu.SemaphoreType.DMA((2,)),
    ],
    compiler_params=pltpu.CompilerParams(use_tc_tiling_on_sc=False,
                                         needs_layout_passes=False),
)(_sc_body)


_NV = 100001
_NVP = 100096


def _nz_body(x_ref, out_ref):
    for r in range(8):
        out_ref[pl.ds(r * _NVP, _NV)] = x_ref[r, :]


def _node_zform(node_t):
    return pl.pallas_call(
        _nz_body,
        grid=(_D // 8,),
        in_specs=[pl.BlockSpec((8, _NV), lambda d: (d, 0))],
        out_specs=pl.BlockSpec((8 * _NVP,), lambda d: (d,)),
        out_shape=jax.ShapeDtypeStruct((_D * _NVP,), jnp.float32),
    )(node_t)


def _sc_node_body(nidx_hbm, zq_hbm, nrowT_hbm, nidx_v, zrow_v, out_v):
    wid = lax.axis_index("s") * _NC + lax.axis_index("c")
    pltpu.sync_copy(zq_hbm.at[wid, :], zrow_v)
    pltpu.sync_copy(nidx_hbm, nidx_v)

    @plsc.parallel_loop(0, _B // 16, step=1, unroll=4)
    def gat(i):
        idx16 = nidx_v[pl.ds(16 * i, 16)]
        out_v[pl.ds(16 * i, 16)] = plsc.load_gather(zrow_v, [idx16])

    pltpu.sync_copy(out_v, nrowT_hbm.at[wid, :])


_sc_node = functools.partial(
    pl.kernel,
    out_type=jax.ShapeDtypeStruct((_D, _B), jnp.float32),
    mesh=_mesh,
    scratch_types=[
        pltpu.VMEM((_B,), jnp.int32),
        pltpu.VMEM((_NVP,), jnp.float32),
        pltpu.VMEM((_B,), jnp.float32),
    ],
    compiler_params=pltpu.CompilerParams(use_tc_tiling_on_sc=False,
                                         needs_layout_passes=False),
)(_sc_node_body)


def _tc_body(fsumt_ref, nrow_ref, fw_ref, fb_ref, nw_ref, nb_ref, out_ref):
    fo = lax.dot_general(fsumt_ref[...], fw_ref[...], (((0,), (1,)), ((), ())),
                         preferred_element_type=jnp.float32)
    no = lax.dot_general(nrow_ref[...], nw_ref[...], (((0,), (1,)), ((), ())),
                         preferred_element_type=jnp.float32)
    out_ref[:, 0:_D] = fo * (1.0 / _L) + fb_ref[...]
    out_ref[:, _D:2 * _D] = no + nb_ref[...]


def kernel(ids, feats, layer_idx, node_table, node_fc_w, node_fc_b,
           feat_table, feat_fc_w, feat_fc_b):
    n_nodes = node_table.shape[0] - 1
    idx = jnp.where(layer_idx > 0, ids,
                    jnp.full_like(ids, n_nodes)).astype(jnp.int32)
    featsT = jnp.swapaxes(feats.astype(jnp.int32), 0, 1)
    ftabT = lax.bitcast_convert_type(
        jnp.swapaxes(feat_table.astype(jnp.bfloat16), 0, 1)
        .reshape(_D // 2, 2, _FV).swapaxes(1, 2),
        jnp.int32)
    node_t = jnp.swapaxes(node_table, 0, 1)
    zq = _node_zform(node_t).reshape(_D, _NVP)
    fsumT = _sc_pool(featsT, ftabT)
    nrow = _sc_node(idx, zq)
    out = pl.pallas_call(
        _tc_body,
        out_shape=jax.ShapeDtypeStruct((_B, 2 * _D), jnp.float32),
    )(fsumT, nrow, feat_fc_w, feat_fc_b.reshape(1, _D),
      node_fc_w, node_fc_b.reshape(1, _D))
    return out
